# baseline (speedup 1.0000x reference)
import jax
import jax.numpy as jnp
from jax.experimental import pallas as pl
from jax.experimental.pallas import tpu as pltpu

L = 16
K = 11
H = 6
H5 = 14
PAD5 = 16


def _tri_dot(t, m_ref):
    f32 = jnp.float32
    u0 = jnp.dot(t[:, :256], m_ref[:256, :256], preferred_element_type=f32)
    u1 = jnp.dot(t[:, :512], m_ref[:512, 256:512], preferred_element_type=f32)
    u2 = jnp.dot(t, m_ref[:, 512:], preferred_element_type=f32)
    return jnp.concatenate([u0, u1, u2], axis=1)


def _conv_stack_kernel(x_ref, m1_ref, b1_ref, m2_ref, b2_ref, m3_ref,
                       b3_ref, m4_ref, b4_ref, m5_ref, b5_ref, out_ref):
    f32, bf16 = jnp.float32, jnp.bfloat16
    xb = x_ref[...][:, 0]
    u = jax.lax.dot_general(xb, m1_ref[...], (((1,), (0,)), ((), ())),
                            preferred_element_type=f32)
    kb, w, hc = u.shape
    t = jnp.tanh(u.reshape(kb * w, hc) + b1_ref[...])
    t = jnp.tanh(_tri_dot(t.astype(bf16), m2_ref) + b2_ref[...])
    t = jnp.tanh(_tri_dot(t.astype(bf16), m3_ref) + b3_ref[...])
    t = jnp.tanh(_tri_dot(t.astype(bf16), m4_ref) + b4_ref[...])
    out_ref[...] = jnp.tanh(jnp.dot(t.astype(bf16), m5_ref[...],
                                    preferred_element_type=f32) + b5_ref[...])


def _head_kernel(y_ref, w_ref, b_ref, v_ref, c_ref, out_ref):
    h = jnp.tanh(jnp.dot(y_ref[...], w_ref[...],
                         preferred_element_type=jnp.float32) + b_ref[...])
    out_ref[...] = (jnp.sum(h * v_ref[...], axis=-1, keepdims=True)
                    + c_ref[...])


def _banded_weights(w1, b1, w2, b2, w3, b3, w4, b4, w5, b5):
    C = w1.shape[0]
    w1k = jnp.transpose(w1[:, 0, :, 0], (1, 0))
    w5k = jnp.transpose(w5[0, :, :, 0], (1, 0))

    r = jnp.arange(L)[:, None]
    h = jnp.arange(H)[None, :]
    w1p = jnp.concatenate([jnp.zeros((H - 1, C), jnp.float32), w1k,
                           jnp.zeros((L - K, C), jnp.float32)])
    m1 = w1p[(r - h) + (H - 1)].reshape(L, H * C)

    def banded(w):
        wk = jnp.transpose(w[:, :, :, 0], (2, 1, 0))
        wkp = jnp.concatenate([wk, jnp.zeros((H, C, C), jnp.float32)])
        j = jnp.arange(H)[:, None]
        hh = jnp.arange(H)[None, :]
        blocks = wkp[(K - 1) + j - hh]
        return jnp.transpose(blocks, (0, 2, 1, 3)).reshape(H * C, H * C)

    jj = jnp.arange(H)[:, None]
    ss = jnp.arange(PAD5)[None, :]
    w5p = jnp.concatenate([jnp.zeros((H - 1, C), jnp.float32), w5k,
                           jnp.zeros((PAD5 - 3, C), jnp.float32)])
    m5 = w5p[((K - 1) + jj - ss) + (H - 1)]
    m5 = jnp.where((ss < H5)[..., None], m5, 0.0)
    m5 = jnp.transpose(m5, (0, 2, 1)).reshape(H * C, PAD5)

    b_row = lambda b: jnp.tile(b, H)[None, :]
    b5r = jnp.broadcast_to(b5.reshape(1, 1), (1, PAD5))
    bf16 = jnp.bfloat16
    return (m1, b_row(b1), banded(w2).astype(bf16), b_row(b2),
            banded(w3).astype(bf16), b_row(b3), banded(w4).astype(bf16),
            b_row(b4), m5.astype(bf16), b5r)


def _forward(x, w1, b1, w2, b2, w3, b3, w4, b4, w5, b5,
             fc1_w, fc1_b, fc2_w, fc2_b):
    B, _, _, W = x.shape
    C = w1.shape[0]
    N = B * W
    HC = H * C

    mats = _banded_weights(w1, b1, w2, b2, w3, b3, w4, b4, w5, b5)

    tb = 2048 if N % 2048 == 0 else W
    kb = tb // W
    wspec = lambda shape: pl.BlockSpec(shape, lambda i: (0,) * len(shape))
    y = pl.pallas_call(
        _conv_stack_kernel,
        out_shape=jax.ShapeDtypeStruct((N, PAD5), jnp.float32),
        grid=(N // tb,),
        in_specs=[
            pl.BlockSpec((kb, 1, L, W), lambda i: (i, 0, 0, 0)),
            wspec((L, HC)), wspec((1, HC)),
            wspec((HC, HC)), wspec((1, HC)),
            wspec((HC, HC)), wspec((1, HC)),
            wspec((HC, HC)), wspec((1, HC)),
            wspec((HC, PAD5)), wspec((1, PAD5)),
        ],
        out_specs=pl.BlockSpec((tb, PAD5), lambda i: (i, 0)),
        compiler_params=pltpu.CompilerParams(
            dimension_semantics=("parallel",)),
    )(x, *mats)

    NH = 128
    g = fc1_w.reshape(-1, L, W)[:, 2:, :]
    g = jnp.pad(g, ((0, NH - g.shape[0]), (0, PAD5 - H5), (0, 0)))
    fc1p = jnp.transpose(g, (2, 1, 0)).reshape(W * PAD5, NH)
    b1h = jnp.pad(fc1_b, (0, NH - fc1_b.shape[0]))[None, :]
    v = jnp.pad(fc2_w[0], (0, NH - fc2_w.shape[1]))[None, :]

    tb2 = 256 if B % 256 == 0 else B
    out = pl.pallas_call(
        _head_kernel,
        out_shape=jax.ShapeDtypeStruct((B, 1), jnp.float32),
        grid=(B // tb2,),
        in_specs=[
            pl.BlockSpec((tb2, W * PAD5), lambda i: (i, 0)),
            wspec((W * PAD5, NH)), wspec((1, NH)), wspec((1, NH)),
            wspec((1, 1)),
        ],
        out_specs=pl.BlockSpec((tb2, 1), lambda i: (i, 0)),
        compiler_params=pltpu.CompilerParams(
            dimension_semantics=("parallel",)),
    )(y.reshape(B, W * PAD5), fc1p, b1h, v, fc2_b.reshape(1, 1))
    return out


def kernel(x, w1, b1, w2, b2, w3, b3, w4, b4, w5, b5,
           fc1_w, fc1_b, fc2_w, fc2_b):
    args = (x, w1, b1, w2, b2, w3, b3, w4, b4, w5, b5,
            fc1_w, fc1_b, fc2_w, fc2_b)
    devs = jax.devices()
    if len(devs) < 2 or x.shape[0] % 2 != 0:
        return _forward(*args)
    mesh = jax.sharding.Mesh(devs[:2], ("d",))
    P = jax.sharding.PartitionSpec
    x = jax.lax.with_sharding_constraint(
        x, jax.sharding.NamedSharding(mesh, P("d")))
    in_specs = (P("d"),) + (P(),) * 14
    f = jax.shard_map(_forward, mesh=mesh, in_specs=in_specs,
                      out_specs=P("d"), check_vma=False)
    return f(x, *args[1:])


# speedup vs baseline: 1.0663x; 1.0663x over previous
import jax
import jax.numpy as jnp
from jax.experimental import pallas as pl
from jax.experimental.pallas import tpu as pltpu

L = 16
K = 11
H = 6
H5 = 14
PAD5 = 16


def _tri_dot(t, m_ref):
    f32 = jnp.float32
    u0 = jnp.dot(t[:, :256], m_ref[:256, :256], preferred_element_type=f32)
    u1 = jnp.dot(t[:, :512], m_ref[:512, 256:512], preferred_element_type=f32)
    u2 = jnp.dot(t, m_ref[:, 512:], preferred_element_type=f32)
    return jnp.concatenate([u0, u1, u2], axis=1)


def _conv_stack_kernel(x_ref, m1_ref, b1_ref, m2_ref, b2_ref, m3_ref,
                       b3_ref, m4_ref, b4_ref, m5_ref, b5_ref, out_ref):
    f32, bf16 = jnp.float32, jnp.bfloat16
    xb = x_ref[...][:, 0]
    u = jax.lax.dot_general(xb, m1_ref[...], (((1,), (0,)), ((), ())),
                            preferred_element_type=f32)
    kb, w, hc = u.shape
    t = jnp.tanh(u.reshape(kb * w, hc) + b1_ref[...])
    t = jnp.tanh(_tri_dot(t.astype(bf16), m2_ref) + b2_ref[...])
    t = jnp.tanh(_tri_dot(t.astype(bf16), m3_ref) + b3_ref[...])
    t = jnp.tanh(_tri_dot(t.astype(bf16), m4_ref) + b4_ref[...])
    out_ref[...] = jnp.tanh(jnp.dot(t.astype(bf16), m5_ref[...],
                                    preferred_element_type=f32) + b5_ref[...])


def _head_kernel(y_ref, w_ref, b_ref, v_ref, c_ref, out_ref):
    h = jnp.tanh(jnp.dot(y_ref[...], w_ref[...],
                         preferred_element_type=jnp.float32) + b_ref[...])
    out_ref[...] = (jnp.sum(h * v_ref[...], axis=-1, keepdims=True)
                    + c_ref[...])


def _banded_weights(w1, b1, w2, b2, w3, b3, w4, b4, w5, b5):
    C = w1.shape[0]
    w1k = jnp.transpose(w1[:, 0, :, 0], (1, 0))
    w5k = jnp.transpose(w5[0, :, :, 0], (1, 0))

    r = jnp.arange(L)[:, None]
    h = jnp.arange(H)[None, :]
    w1p = jnp.concatenate([jnp.zeros((H - 1, C), jnp.float32), w1k,
                           jnp.zeros((L - K, C), jnp.float32)])
    m1 = w1p[(r - h) + (H - 1)].reshape(L, H * C)

    def banded(w):
        wk = jnp.transpose(w[:, :, :, 0], (2, 1, 0))
        wkp = jnp.concatenate([wk, jnp.zeros((H, C, C), jnp.float32)])
        j = jnp.arange(H)[:, None]
        hh = jnp.arange(H)[None, :]
        blocks = wkp[(K - 1) + j - hh]
        return jnp.transpose(blocks, (0, 2, 1, 3)).reshape(H * C, H * C)

    jj = jnp.arange(H)[:, None]
    ss = jnp.arange(PAD5)[None, :]
    w5p = jnp.concatenate([jnp.zeros((H - 1, C), jnp.float32), w5k,
                           jnp.zeros((PAD5 - 3, C), jnp.float32)])
    m5 = w5p[((K - 1) + jj - ss) + (H - 1)]
    m5 = jnp.where((ss < H5)[..., None], m5, 0.0)
    m5 = jnp.transpose(m5, (0, 2, 1)).reshape(H * C, PAD5)

    b_row = lambda b: jnp.tile(b, H)[None, :]
    b5r = jnp.broadcast_to(b5.reshape(1, 1), (1, PAD5))
    bf16 = jnp.bfloat16
    return (m1, b_row(b1), banded(w2).astype(bf16), b_row(b2),
            banded(w3).astype(bf16), b_row(b3), banded(w4).astype(bf16),
            b_row(b4), m5.astype(bf16), b5r)


def _forward(x, w1, b1, w2, b2, w3, b3, w4, b4, w5, b5,
             fc1_w, fc1_b, fc2_w, fc2_b):
    B, _, _, W = x.shape
    C = w1.shape[0]
    N = B * W
    HC = H * C

    mats = _banded_weights(w1, b1, w2, b2, w3, b3, w4, b4, w5, b5)

    tb = 4096 if N % 4096 == 0 else W
    kb = tb // W
    wspec = lambda shape: pl.BlockSpec(shape, lambda i: (0,) * len(shape))
    y = pl.pallas_call(
        _conv_stack_kernel,
        out_shape=jax.ShapeDtypeStruct((N, PAD5), jnp.float32),
        grid=(N // tb,),
        in_specs=[
            pl.BlockSpec((kb, 1, L, W), lambda i: (i, 0, 0, 0)),
            wspec((L, HC)), wspec((1, HC)),
            wspec((HC, HC)), wspec((1, HC)),
            wspec((HC, HC)), wspec((1, HC)),
            wspec((HC, HC)), wspec((1, HC)),
            wspec((HC, PAD5)), wspec((1, PAD5)),
        ],
        out_specs=pl.BlockSpec((tb, PAD5), lambda i: (i, 0)),
        compiler_params=pltpu.CompilerParams(
            dimension_semantics=("parallel",)),
    )(x, *mats)

    NH = 128
    g = fc1_w.reshape(-1, L, W)[:, 2:, :]
    g = jnp.pad(g, ((0, NH - g.shape[0]), (0, PAD5 - H5), (0, 0)))
    fc1p = jnp.transpose(g, (2, 1, 0)).reshape(W * PAD5, NH)
    b1h = jnp.pad(fc1_b, (0, NH - fc1_b.shape[0]))[None, :]
    v = jnp.pad(fc2_w[0], (0, NH - fc2_w.shape[1]))[None, :]

    tb2 = 256 if B % 256 == 0 else B
    out = pl.pallas_call(
        _head_kernel,
        out_shape=jax.ShapeDtypeStruct((B, 1), jnp.float32),
        grid=(B // tb2,),
        in_specs=[
            pl.BlockSpec((tb2, W * PAD5), lambda i: (i, 0)),
            wspec((W * PAD5, NH)), wspec((1, NH)), wspec((1, NH)),
            wspec((1, 1)),
        ],
        out_specs=pl.BlockSpec((tb2, 1), lambda i: (i, 0)),
        compiler_params=pltpu.CompilerParams(
            dimension_semantics=("parallel",)),
    )(y.reshape(B, W * PAD5), fc1p, b1h, v, fc2_b.reshape(1, 1))
    return out


def kernel(x, w1, b1, w2, b2, w3, b3, w4, b4, w5, b5,
           fc1_w, fc1_b, fc2_w, fc2_b):
    args = (x, w1, b1, w2, b2, w3, b3, w4, b4, w5, b5,
            fc1_w, fc1_b, fc2_w, fc2_b)
    devs = jax.devices()
    if len(devs) < 2 or x.shape[0] % 2 != 0:
        return _forward(*args)
    mesh = jax.sharding.Mesh(devs[:2], ("d",))
    P = jax.sharding.PartitionSpec
    x = jax.lax.with_sharding_constraint(
        x, jax.sharding.NamedSharding(mesh, P("d")))
    in_specs = (P("d"),) + (P(),) * 14
    f = jax.shard_map(_forward, mesh=mesh, in_specs=in_specs,
                      out_specs=P("d"), check_vma=False)
    return f(x, *args[1:])


# speedup vs baseline: 1.0866x; 1.0191x over previous
import jax
import jax.numpy as jnp
from jax.experimental import pallas as pl
from jax.experimental.pallas import tpu as pltpu

L = 16
K = 11
H = 6
H5 = 14
PAD5 = 16


def _tri_dot(t, m_ref):
    f32 = jnp.float32
    u0 = jnp.dot(t[:, :256], m_ref[:256, :256], preferred_element_type=f32)
    u1 = jnp.dot(t[:, :512], m_ref[:512, 256:512], preferred_element_type=f32)
    u2 = jnp.dot(t, m_ref[:, 512:], preferred_element_type=f32)
    return jnp.concatenate([u0, u1, u2], axis=1)


def _conv_stack_kernel(x_ref, m1_ref, b1_ref, m2_ref, b2_ref, m3_ref,
                       b3_ref, m4_ref, b4_ref, m5_ref, b5_ref, out_ref):
    f32, bf16 = jnp.float32, jnp.bfloat16
    xb = x_ref[...][:, 0]
    u = jax.lax.dot_general(xb, m1_ref[...], (((1,), (0,)), ((), ())),
                            preferred_element_type=f32)
    kb, w, hc = u.shape
    t = jnp.tanh(u.reshape(kb * w, hc) + b1_ref[...])
    t = jnp.tanh(_tri_dot(t.astype(bf16), m2_ref) + b2_ref[...])
    t = jnp.tanh(_tri_dot(t.astype(bf16), m3_ref) + b3_ref[...])
    t = jnp.tanh(_tri_dot(t.astype(bf16), m4_ref) + b4_ref[...])
    out_ref[...] = jnp.tanh(jnp.dot(t.astype(bf16), m5_ref[...],
                                    preferred_element_type=f32) + b5_ref[...])


def _head_kernel(y_ref, w_ref, b_ref, v_ref, c_ref, out_ref):
    h = jnp.tanh(jnp.dot(y_ref[...], w_ref[...],
                         preferred_element_type=jnp.float32) + b_ref[...])
    out_ref[...] = (jnp.sum(h * v_ref[...], axis=-1, keepdims=True)
                    + c_ref[...])


def _banded_weights(w1, b1, w2, b2, w3, b3, w4, b4, w5, b5):
    C = w1.shape[0]
    w1k = jnp.transpose(w1[:, 0, :, 0], (1, 0))
    w5k = jnp.transpose(w5[0, :, :, 0], (1, 0))

    r = jnp.arange(L)[:, None]
    h = jnp.arange(H)[None, :]
    w1p = jnp.concatenate([jnp.zeros((H - 1, C), jnp.float32), w1k,
                           jnp.zeros((L - K, C), jnp.float32)])
    m1 = w1p[(r - h) + (H - 1)].reshape(L, H * C)

    def banded(w):
        wk = jnp.transpose(w[:, :, :, 0], (2, 1, 0))
        wkp = jnp.concatenate([wk, jnp.zeros((H, C, C), jnp.float32)])
        j = jnp.arange(H)[:, None]
        hh = jnp.arange(H)[None, :]
        blocks = wkp[(K - 1) + j - hh]
        return jnp.transpose(blocks, (0, 2, 1, 3)).reshape(H * C, H * C)

    jj = jnp.arange(H)[:, None]
    ss = jnp.arange(PAD5)[None, :]
    w5p = jnp.concatenate([jnp.zeros((H - 1, C), jnp.float32), w5k,
                           jnp.zeros((PAD5 - 3, C), jnp.float32)])
    m5 = w5p[((K - 1) + jj - ss) + (H - 1)]
    m5 = jnp.where((ss < H5)[..., None], m5, 0.0)
    m5 = jnp.transpose(m5, (0, 2, 1)).reshape(H * C, PAD5)

    b_row = lambda b: jnp.tile(b, H)[None, :]
    b5r = jnp.broadcast_to(b5.reshape(1, 1), (1, PAD5))
    bf16 = jnp.bfloat16
    return (m1, b_row(b1), banded(w2).astype(bf16), b_row(b2),
            banded(w3).astype(bf16), b_row(b3), banded(w4).astype(bf16),
            b_row(b4), m5.astype(bf16), b5r)


def _forward(x, w1, b1, w2, b2, w3, b3, w4, b4, w5, b5,
             fc1_w, fc1_b, fc2_w, fc2_b):
    B, _, _, W = x.shape
    C = w1.shape[0]
    N = B * W
    HC = H * C

    mats = _banded_weights(w1, b1, w2, b2, w3, b3, w4, b4, w5, b5)

    tb = 8192 if N % 8192 == 0 else W
    kb = tb // W
    wspec = lambda shape: pl.BlockSpec(shape, lambda i: (0,) * len(shape))
    y = pl.pallas_call(
        _conv_stack_kernel,
        out_shape=jax.ShapeDtypeStruct((N, PAD5), jnp.float32),
        grid=(N // tb,),
        in_specs=[
            pl.BlockSpec((kb, 1, L, W), lambda i: (i, 0, 0, 0)),
            wspec((L, HC)), wspec((1, HC)),
            wspec((HC, HC)), wspec((1, HC)),
            wspec((HC, HC)), wspec((1, HC)),
            wspec((HC, HC)), wspec((1, HC)),
            wspec((HC, PAD5)), wspec((1, PAD5)),
        ],
        out_specs=pl.BlockSpec((tb, PAD5), lambda i: (i, 0)),
        compiler_params=pltpu.CompilerParams(
            dimension_semantics=("parallel",)),
    )(x, *mats)

    NH = 128
    g = fc1_w.reshape(-1, L, W)[:, 2:, :]
    g = jnp.pad(g, ((0, NH - g.shape[0]), (0, PAD5 - H5), (0, 0)))
    fc1p = jnp.transpose(g, (2, 1, 0)).reshape(W * PAD5, NH)
    b1h = jnp.pad(fc1_b, (0, NH - fc1_b.shape[0]))[None, :]
    v = jnp.pad(fc2_w[0], (0, NH - fc2_w.shape[1]))[None, :]

    tb2 = 256 if B % 256 == 0 else B
    out = pl.pallas_call(
        _head_kernel,
        out_shape=jax.ShapeDtypeStruct((B, 1), jnp.float32),
        grid=(B // tb2,),
        in_specs=[
            pl.BlockSpec((tb2, W * PAD5), lambda i: (i, 0)),
            wspec((W * PAD5, NH)), wspec((1, NH)), wspec((1, NH)),
            wspec((1, 1)),
        ],
        out_specs=pl.BlockSpec((tb2, 1), lambda i: (i, 0)),
        compiler_params=pltpu.CompilerParams(
            dimension_semantics=("parallel",)),
    )(y.reshape(B, W * PAD5), fc1p, b1h, v, fc2_b.reshape(1, 1))
    return out


def kernel(x, w1, b1, w2, b2, w3, b3, w4, b4, w5, b5,
           fc1_w, fc1_b, fc2_w, fc2_b):
    args = (x, w1, b1, w2, b2, w3, b3, w4, b4, w5, b5,
            fc1_w, fc1_b, fc2_w, fc2_b)
    devs = jax.devices()
    if len(devs) < 2 or x.shape[0] % 2 != 0:
        return _forward(*args)
    mesh = jax.sharding.Mesh(devs[:2], ("d",))
    P = jax.sharding.PartitionSpec
    x = jax.lax.with_sharding_constraint(
        x, jax.sharding.NamedSharding(mesh, P("d")))
    in_specs = (P("d"),) + (P(),) * 14
    f = jax.shard_map(_forward, mesh=mesh, in_specs=in_specs,
                      out_specs=P("d"), check_vma=False)
    return f(x, *args[1:])
